# baseline (speedup 1.0000x reference)
import functools
import math

import jax
import jax.numpy as jnp
import numpy as np
from jax import lax
from jax.experimental import pallas as pl
from jax.experimental.pallas import tpu as pltpu
from jax.experimental.pallas import tpu_sc as plsc


def _log_fact(n):
    n = n + 1
    if n < 9:
        return np.log(math.factorial(int(n)))
    return 0.5 * (np.log(2 * np.pi) - np.log(n)) + n * (
        np.log(n + 1.0 / (12 * n - 1.0 / 10 / n)) - 1
    )


def _os_threshold(k, n, pfa):
    def fun(t_os):
        return (
            _log_fact(n)
            - _log_fact(n - k)
            - np.sum(np.log(np.arange(n, n - k, -1) + t_os))
            - np.log(pfa)
        )

    t_max, t_min = 1e32, 1.0
    for _ in range(10000):
        m_n = t_max - fun(t_max) * (t_min - t_max) / (fun(t_min) - fun(t_max))
        f_m_n = fun(m_n)
        if f_m_n == 0 or np.abs(t_max - t_min) < 1e-4:
            return m_n
        if fun(t_max) * f_m_n < 0:
            t_min = m_n
        elif fun(t_min) * f_m_n < 0:
            t_max = m_n
        else:
            break
    raise ValueError("CFAR threshold did not converge")


_OS_N = 16
_K_ORDER = _OS_N * 3 / 4
_PFA = 1e-05
OS_ALPHA = float(np.sqrt(_os_threshold(_K_ORDER, _OS_N, _PFA)))
_OFFS = tuple(range(-10, -2)) + tuple(range(3, 11))
_SCALE = OS_ALPHA / 16.0

_B, _V, _R = 8, 256, 1024
_NC, _NS = 2, 16
_RCHUNK = _R // 4
_COLS = _RCHUNK + 32
_VHALF = _V // 2


def _sort4(a, b, c, d):
    mx, mn = jnp.maximum, jnp.minimum
    h1, l1 = mx(a, b), mn(a, b)
    h2, l2 = mx(c, d), mn(c, d)
    e1, t1 = mx(h1, h2), mn(h1, h2)
    t2, e4 = mx(l1, l2), mn(l1, l2)
    e2, e3 = mx(t1, t2), mn(t1, t2)
    return e1, e2, e3, e4


def _merge44_full(a, b):
    mx, mn = jnp.maximum, jnp.minimum
    a1, a2, a3, a4 = a
    b1, b2, b3, b4 = b
    c1 = mx(a1, b1)
    c2 = mx(mx(mn(a1, b1), a2), b2)
    c3 = mx(mx(mn(a1, b2), mn(a2, b1)), mx(a3, b3))
    c4 = mx(mx(mn(a1, b3), mn(a2, b2)), mx(mn(a3, b1), mx(a4, b4)))
    return c1, c2, c3, c4


def _merge44_4th(a, b):
    mx, mn = jnp.maximum, jnp.minimum
    a1, a2, a3, a4 = a
    b1, b2, b3, b4 = b
    return mx(mx(mn(a1, b3), mn(a2, b2)), mx(mn(a3, b1), mx(a4, b4)))


def _m_pass2(in_slab, ga, ra, rb, m_tile):
    lda = [in_slab[ga, ra, pl.ds(6 + d, 16)] for d in range(8)]
    ldb = [in_slab[ga, rb, pl.ds(6 + d, 16)] for d in range(8)]
    for mb in range(17):
        x0 = 6 + mb * 16
        cura, curb = lda, ldb
        if mb < 16:
            lda = [in_slab[ga, ra, pl.ds(x0 + 16 + d, 16)] for d in range(8)]
            ldb = [in_slab[ga, rb, pl.ds(x0 + 16 + d, 16)] for d in range(8)]
        mma = _merge44_full(_sort4(*cura[0:4]), _sort4(*cura[4:8]))
        mmb = _merge44_full(_sort4(*curb[0:4]), _sort4(*curb[4:8]))
        for i in range(4):
            m_tile[i, pl.ds(x0, 16)] = mma[i]
            m_tile[4 + i, pl.ds(x0, 16)] = mmb[i]


def _final_pass2(m_tile, os_tile, rowa, rowb):
    def ld(jb):
        mla = tuple(m_tile[i, pl.ds(6 + jb * 16, 16)] for i in range(4))
        mra = tuple(m_tile[i, pl.ds(19 + jb * 16, 16)] for i in range(4))
        mlb = tuple(m_tile[4 + i, pl.ds(6 + jb * 16, 16)] for i in range(4))
        mrb = tuple(m_tile[4 + i, pl.ds(19 + jb * 16, 16)] for i in range(4))
        return mla, mra, mlb, mrb

    cur = ld(0)
    for jb in range(_RCHUNK // 16):
        mla, mra, mlb, mrb = cur
        if jb < _RCHUNK // 16 - 1:
            cur = ld(jb + 1)
        os_tile[rowa, pl.ds(jb * 16, 16)] = _merge44_4th(mla, mra)
        os_tile[rowb, pl.ds(jb * 16, 16)] = _merge44_4th(mlb, mrb)


def _cfar_body(data_hbm, out_hbm, in_slab, os_tile, out_slab, m_tile, sem_i, sem_o):
    wid = lax.axis_index("s") * _NC + lax.axis_index("c")
    b = wid // 4
    rc = wid % 4
    r0 = rc * _RCHUNK
    rt0 = 2 * rc
    rtl = jnp.where(rc == 0, 7, 2 * rc - 1)
    rtr = jnp.where(rc == 3, 0, 2 * rc + 2)

    for h in range(2):
        v0 = h * _VHALF
        vgs = pl.ds(h * (_VHALF // 8), _VHALF // 8)
        handles = [
            pltpu.async_copy(
                data_hbm.at[b, vgs, rt0, :, :],
                in_slab.at[:, :, pl.ds(16, 128)],
                sem_i,
            ),
            pltpu.async_copy(
                data_hbm.at[b, vgs, rt0 + 1, :, :],
                in_slab.at[:, :, pl.ds(144, 128)],
                sem_i,
            ),
            pltpu.async_copy(
                data_hbm.at[b, vgs, rtl, :, pl.ds(112, 16)],
                in_slab.at[:, :, pl.ds(0, 16)],
                sem_i,
            ),
            pltpu.async_copy(
                data_hbm.at[b, vgs, rtr, :, pl.ds(0, 16)],
                in_slab.at[:, :, pl.ds(272, 16)],
                sem_i,
            ),
        ]
        for hh in handles:
            hh.wait()

        def row_pair(vi, _):
            ga = vi // 4
            ra = 2 * (vi % 4)
            _m_pass2(in_slab, ga, ra, ra + 1, m_tile)
            _final_pass2(m_tile, os_tile, 10 + v0 + 2 * vi, 11 + v0 + 2 * vi)
            return 0

        lax.fori_loop(0, _VHALF // 2, row_pair, 0)

    for hr in range(10):
        for jb in range(_RCHUNK // 16):
            os_tile[hr, pl.ds(jb * 16, 16)] = os_tile[256 + hr, pl.ds(jb * 16, 16)]
    for hr in range(11):
        for jb in range(_RCHUNK // 16):
            os_tile[266 + hr, pl.ds(jb * 16, 16)] = os_tile[10 + hr, pl.ds(jb * 16, 16)]

    def _drain_out():
        for j in range(2):
            pltpu.make_async_copy(
                out_slab.at[:, :, pl.ds(128 * j, 128)],
                out_hbm.at[b, pl.ds(0, 8), rt0 + j, :, :],
                sem_o,
            ).wait()

    def ca_group(g, _):
        vg = g * 64

        @pl.when(g > 0)
        def _():
            _drain_out()

        for jb in range(_RCHUNK // 16):
            cb = pl.ds(jb * 16, 16)
            acc = None
            for off in _OFFS:
                x = os_tile[vg + 10 + off, cb]
                acc = x if acc is None else acc + x

            def ca_rows(vv, s):
                v = vg + 4 * vv
                oq = vv // 2
                orr = 4 * (vv % 2)
                for u in range(4):
                    out_slab[oq, orr + u, cb] = s * _SCALE
                    d = (os_tile[v + u + 21, cb] + os_tile[v + u + 8, cb]) - (
                        os_tile[v + u, cb] + os_tile[v + u + 13, cb]
                    )
                    s = s + d
                return s

            lax.fori_loop(0, 16, ca_rows, acc)
        for j in range(2):
            pltpu.async_copy(
                out_slab.at[:, :, pl.ds(128 * j, 128)],
                out_hbm.at[b, pl.ds(g * 8, 8), rt0 + j, :, :],
                sem_o,
            )
        return 0

    lax.fori_loop(0, 4, ca_group, 0)
    _drain_out()


@jax.jit
def kernel(data):
    mesh = plsc.VectorSubcoreMesh(core_axis_name="c", subcore_axis_name="s")
    run = functools.partial(
        pl.kernel,
        mesh=mesh,
        out_type=jax.ShapeDtypeStruct((_B, _V // 8, _R // 128, 8, 128), jnp.float32),
        scratch_types=[
            pltpu.VMEM((_VHALF // 8, 8, _COLS), jnp.float32),
            pltpu.VMEM((_V + 21, _RCHUNK), jnp.float32),
            pltpu.VMEM((8, 8, _RCHUNK), jnp.float32),
            pltpu.VMEM((8, _COLS), jnp.float32),
            pltpu.SemaphoreType.DMA,
            pltpu.SemaphoreType.DMA,
        ],
        compiler_params=pltpu.CompilerParams(use_tc_tiling_on_sc=False),
    )(_cfar_body)
    data5 = data.reshape(_B, _V // 8, 8, _R // 128, 128).transpose(0, 1, 3, 2, 4)
    out5 = run(data5)
    return out5.transpose(0, 1, 3, 2, 4).reshape(_B, _V, _R)


# speedup vs baseline: 1.0187x; 1.0187x over previous
import functools
import math

import jax
import jax.numpy as jnp
import numpy as np
from jax import lax
from jax.experimental import pallas as pl
from jax.experimental.pallas import tpu as pltpu
from jax.experimental.pallas import tpu_sc as plsc


def _log_fact(n):
    n = n + 1
    if n < 9:
        return np.log(math.factorial(int(n)))
    return 0.5 * (np.log(2 * np.pi) - np.log(n)) + n * (
        np.log(n + 1.0 / (12 * n - 1.0 / 10 / n)) - 1
    )


def _os_threshold(k, n, pfa):
    def fun(t_os):
        return (
            _log_fact(n)
            - _log_fact(n - k)
            - np.sum(np.log(np.arange(n, n - k, -1) + t_os))
            - np.log(pfa)
        )

    t_max, t_min = 1e32, 1.0
    for _ in range(10000):
        m_n = t_max - fun(t_max) * (t_min - t_max) / (fun(t_min) - fun(t_max))
        f_m_n = fun(m_n)
        if f_m_n == 0 or np.abs(t_max - t_min) < 1e-4:
            return m_n
        if fun(t_max) * f_m_n < 0:
            t_min = m_n
        elif fun(t_min) * f_m_n < 0:
            t_max = m_n
        else:
            break
    raise ValueError("CFAR threshold did not converge")


_OS_N = 16
_K_ORDER = _OS_N * 3 / 4
_PFA = 1e-05
OS_ALPHA = float(np.sqrt(_os_threshold(_K_ORDER, _OS_N, _PFA)))
_OFFS = tuple(range(-10, -2)) + tuple(range(3, 11))
_SCALE = OS_ALPHA / 16.0

_B, _V, _R = 8, 256, 1024
_NC, _NS = 2, 16
_RCHUNK = _R // 4
_COLS = _RCHUNK + 32
_VHALF = _V // 2


def _sort4(a, b, c, d):
    mx, mn = jnp.maximum, jnp.minimum
    h1, l1 = mx(a, b), mn(a, b)
    h2, l2 = mx(c, d), mn(c, d)
    e1, t1 = mx(h1, h2), mn(h1, h2)
    t2, e4 = mx(l1, l2), mn(l1, l2)
    e2, e3 = mx(t1, t2), mn(t1, t2)
    return e1, e2, e3, e4


def _merge44_full(a, b):
    mx, mn = jnp.maximum, jnp.minimum
    a1, a2, a3, a4 = a
    b1, b2, b3, b4 = b
    c1 = mx(a1, b1)
    c2 = mx(mx(mn(a1, b1), a2), b2)
    c3 = mx(mx(mn(a1, b2), mn(a2, b1)), mx(a3, b3))
    c4 = mx(mx(mn(a1, b3), mn(a2, b2)), mx(mn(a3, b1), mx(a4, b4)))
    return c1, c2, c3, c4


def _merge44_4th(a, b):
    mx, mn = jnp.maximum, jnp.minimum
    a1, a2, a3, a4 = a
    b1, b2, b3, b4 = b
    return mx(mx(mn(a1, b3), mn(a2, b2)), mx(mn(a3, b1), mx(a4, b4)))


def _m_pass2(in_slab, ga, ra, rb, m_tile):
    lda = [in_slab[ga, ra, pl.ds(6 + d, 16)] for d in range(8)]
    ldb = [in_slab[ga, rb, pl.ds(6 + d, 16)] for d in range(8)]
    for mb in range(17):
        x0 = 6 + mb * 16
        cura, curb = lda, ldb
        if mb < 16:
            lda = [in_slab[ga, ra, pl.ds(x0 + 16 + d, 16)] for d in range(8)]
            ldb = [in_slab[ga, rb, pl.ds(x0 + 16 + d, 16)] for d in range(8)]
        mma = _merge44_full(_sort4(*cura[0:4]), _sort4(*cura[4:8]))
        mmb = _merge44_full(_sort4(*curb[0:4]), _sort4(*curb[4:8]))
        for i in range(4):
            m_tile[i, pl.ds(x0, 16)] = mma[i]
            m_tile[4 + i, pl.ds(x0, 16)] = mmb[i]


def _final_pass2(m_tile, os_tile, rowa, rowb):
    def ld(jb):
        mla = tuple(m_tile[i, pl.ds(6 + jb * 16, 16)] for i in range(4))
        mra = tuple(m_tile[i, pl.ds(19 + jb * 16, 16)] for i in range(4))
        mlb = tuple(m_tile[4 + i, pl.ds(6 + jb * 16, 16)] for i in range(4))
        mrb = tuple(m_tile[4 + i, pl.ds(19 + jb * 16, 16)] for i in range(4))
        return mla, mra, mlb, mrb

    cur = ld(0)
    for jb in range(_RCHUNK // 16):
        mla, mra, mlb, mrb = cur
        if jb < _RCHUNK // 16 - 1:
            cur = ld(jb + 1)
        os_tile[rowa, pl.ds(jb * 16, 16)] = _merge44_4th(mla, mra)
        os_tile[rowb, pl.ds(jb * 16, 16)] = _merge44_4th(mlb, mrb)


def _cfar_body(data_hbm, out_hbm, in_slab, os_tile, out_slab, m_tile, sem_i, sem_o):
    wid = lax.axis_index("s") * _NC + lax.axis_index("c")
    b = wid // 4
    rc = wid % 4
    r0 = rc * _RCHUNK
    rt0 = 2 * rc
    rtl = jnp.where(rc == 0, 7, 2 * rc - 1)
    rtr = jnp.where(rc == 3, 0, 2 * rc + 2)

    for h in range(2):
        v0 = h * _VHALF
        vgs = pl.ds(h * (_VHALF // 8), _VHALF // 8)
        handles = [
            pltpu.async_copy(
                data_hbm.at[b, vgs, rt0, :, :],
                in_slab.at[:, :, pl.ds(16, 128)],
                sem_i,
            ),
            pltpu.async_copy(
                data_hbm.at[b, vgs, rt0 + 1, :, :],
                in_slab.at[:, :, pl.ds(144, 128)],
                sem_i,
            ),
            pltpu.async_copy(
                data_hbm.at[b, vgs, rtl, :, pl.ds(112, 16)],
                in_slab.at[:, :, pl.ds(0, 16)],
                sem_i,
            ),
            pltpu.async_copy(
                data_hbm.at[b, vgs, rtr, :, pl.ds(0, 16)],
                in_slab.at[:, :, pl.ds(272, 16)],
                sem_i,
            ),
        ]
        for hh in handles:
            hh.wait()

        def row_pair(vi, _):
            ga = vi // 4
            ra = 2 * (vi % 4)
            _m_pass2(in_slab, ga, ra, ra + 1, m_tile)
            _final_pass2(m_tile, os_tile, 10 + v0 + 2 * vi, 11 + v0 + 2 * vi)
            return 0

        lax.fori_loop(0, _VHALF // 2, row_pair, 0)

    for hr in range(10):
        for jb in range(_RCHUNK // 16):
            os_tile[hr, pl.ds(jb * 16, 16)] = os_tile[256 + hr, pl.ds(jb * 16, 16)]
    for hr in range(11):
        for jb in range(_RCHUNK // 16):
            os_tile[266 + hr, pl.ds(jb * 16, 16)] = os_tile[10 + hr, pl.ds(jb * 16, 16)]

    out_handles = []
    for g in range(4):
        vg = g * 64
        for hh in out_handles:
            hh.wait()
        out_handles = []

        def ca_col_block(jb, _):
            cb = pl.ds(jb * 16, 16)
            acc = None
            for off in _OFFS:
                x = os_tile[vg + 10 + off, cb]
                acc = x if acc is None else acc + x

            def ca_rows(vv, s):
                v = vg + 4 * vv
                oq = vv // 2
                orr = 4 * (vv % 2)
                for u in range(4):
                    out_slab[oq, orr + u, cb] = s * _SCALE
                    d = (os_tile[v + u + 21, cb] + os_tile[v + u + 8, cb]) - (
                        os_tile[v + u, cb] + os_tile[v + u + 13, cb]
                    )
                    s = s + d
                return s

            lax.fori_loop(0, 16, ca_rows, acc)
            return 0

        lax.fori_loop(0, _RCHUNK // 16, ca_col_block, 0)
        for j in range(2):
            out_handles.append(
                pltpu.async_copy(
                    out_slab.at[:, :, pl.ds(128 * j, 128)],
                    out_hbm.at[b, pl.ds(g * 8, 8), rt0 + j, :, :],
                    sem_o,
                )
            )
    for hh in out_handles:
        hh.wait()


@jax.jit
def kernel(data):
    mesh = plsc.VectorSubcoreMesh(core_axis_name="c", subcore_axis_name="s")
    run = functools.partial(
        pl.kernel,
        mesh=mesh,
        out_type=jax.ShapeDtypeStruct((_B, _V // 8, _R // 128, 8, 128), jnp.float32),
        scratch_types=[
            pltpu.VMEM((_VHALF // 8, 8, _COLS), jnp.float32),
            pltpu.VMEM((_V + 21, _RCHUNK), jnp.float32),
            pltpu.VMEM((8, 8, _RCHUNK), jnp.float32),
            pltpu.VMEM((8, _COLS), jnp.float32),
            pltpu.SemaphoreType.DMA,
            pltpu.SemaphoreType.DMA,
        ],
        compiler_params=pltpu.CompilerParams(use_tc_tiling_on_sc=False),
    )(_cfar_body)
    data5 = data.reshape(_B, _V // 8, 8, _R // 128, 128).transpose(0, 1, 3, 2, 4)
    out5 = run(data5)
    return out5.transpose(0, 1, 3, 2, 4).reshape(_B, _V, _R)


# speedup vs baseline: 1.0976x; 1.0774x over previous
import functools
import math

import jax
import jax.numpy as jnp
import numpy as np
from jax import lax
from jax.experimental import pallas as pl
from jax.experimental.pallas import tpu as pltpu
from jax.experimental.pallas import tpu_sc as plsc


def _log_fact(n):
    n = n + 1
    if n < 9:
        return np.log(math.factorial(int(n)))
    return 0.5 * (np.log(2 * np.pi) - np.log(n)) + n * (
        np.log(n + 1.0 / (12 * n - 1.0 / 10 / n)) - 1
    )


def _os_threshold(k, n, pfa):
    def fun(t_os):
        return (
            _log_fact(n)
            - _log_fact(n - k)
            - np.sum(np.log(np.arange(n, n - k, -1) + t_os))
            - np.log(pfa)
        )

    t_max, t_min = 1e32, 1.0
    for _ in range(10000):
        m_n = t_max - fun(t_max) * (t_min - t_max) / (fun(t_min) - fun(t_max))
        f_m_n = fun(m_n)
        if f_m_n == 0 or np.abs(t_max - t_min) < 1e-4:
            return m_n
        if fun(t_max) * f_m_n < 0:
            t_min = m_n
        elif fun(t_min) * f_m_n < 0:
            t_max = m_n
        else:
            break
    raise ValueError("CFAR threshold did not converge")


_OS_N = 16
_K_ORDER = _OS_N * 3 / 4
_PFA = 1e-05
OS_ALPHA = float(np.sqrt(_os_threshold(_K_ORDER, _OS_N, _PFA)))
_OFFS = tuple(range(-10, -2)) + tuple(range(3, 11))
_SCALE = OS_ALPHA / 16.0

_B, _V, _R = 8, 256, 1024
_NC, _NS = 2, 16
_RCHUNK = _R // 4
_COLS = _RCHUNK + 32
_VHALF = _V // 2


def _sort4(a, b, c, d):
    mx, mn = jnp.maximum, jnp.minimum
    h1, l1 = mx(a, b), mn(a, b)
    h2, l2 = mx(c, d), mn(c, d)
    e1, t1 = mx(h1, h2), mn(h1, h2)
    t2, e4 = mx(l1, l2), mn(l1, l2)
    e2, e3 = mx(t1, t2), mn(t1, t2)
    return e1, e2, e3, e4


def _merge44_full(a, b):
    mx, mn = jnp.maximum, jnp.minimum
    a1, a2, a3, a4 = a
    b1, b2, b3, b4 = b
    c1 = mx(a1, b1)
    c2 = mx(mx(mn(a1, b1), a2), b2)
    c3 = mx(mx(mn(a1, b2), mn(a2, b1)), mx(a3, b3))
    c4 = mx(mx(mn(a1, b3), mn(a2, b2)), mx(mn(a3, b1), mx(a4, b4)))
    return c1, c2, c3, c4


def _merge44_4th(a, b):
    mx, mn = jnp.maximum, jnp.minimum
    a1, a2, a3, a4 = a
    b1, b2, b3, b4 = b
    return mx(mx(mn(a1, b3), mn(a2, b2)), mx(mn(a3, b1), mx(a4, b4)))


def _m_pass2(in_slab, ga, ra, rb, m_tile):
    lda = [in_slab[ga, ra, pl.ds(6 + d, 16)] for d in range(8)]
    ldb = [in_slab[ga, rb, pl.ds(6 + d, 16)] for d in range(8)]
    for mb in range(17):
        x0 = 6 + mb * 16
        cura, curb = lda, ldb
        if mb < 16:
            lda = [in_slab[ga, ra, pl.ds(x0 + 16 + d, 16)] for d in range(8)]
            ldb = [in_slab[ga, rb, pl.ds(x0 + 16 + d, 16)] for d in range(8)]
        mma = _merge44_full(_sort4(*cura[0:4]), _sort4(*cura[4:8]))
        mmb = _merge44_full(_sort4(*curb[0:4]), _sort4(*curb[4:8]))
        for i in range(4):
            m_tile[i, pl.ds(x0, 16)] = mma[i]
            m_tile[4 + i, pl.ds(x0, 16)] = mmb[i]


def _final_pass2(m_tile, os_tile, rowa, rowb):
    def ld(jb):
        mla = tuple(m_tile[i, pl.ds(6 + jb * 16, 16)] for i in range(4))
        mra = tuple(m_tile[i, pl.ds(19 + jb * 16, 16)] for i in range(4))
        mlb = tuple(m_tile[4 + i, pl.ds(6 + jb * 16, 16)] for i in range(4))
        mrb = tuple(m_tile[4 + i, pl.ds(19 + jb * 16, 16)] for i in range(4))
        return mla, mra, mlb, mrb

    cur = ld(0)
    for jb in range(_RCHUNK // 16):
        mla, mra, mlb, mrb = cur
        if jb < _RCHUNK // 16 - 1:
            cur = ld(jb + 1)
        os_tile[rowa, pl.ds(jb * 16, 16)] = _merge44_4th(mla, mra)
        os_tile[rowb, pl.ds(jb * 16, 16)] = _merge44_4th(mlb, mrb)


def _cfar_body(data_hbm, out_hbm, in_slab, os_tile, out_slab, m_tile, sem_i, sem_o):
    wid = lax.axis_index("s") * _NC + lax.axis_index("c")
    b = wid // 4
    rc = wid % 4
    r0 = rc * _RCHUNK
    rt0 = 2 * rc
    rtl = jnp.where(rc == 0, 7, 2 * rc - 1)
    rtr = jnp.where(rc == 3, 0, 2 * rc + 2)

    for h in range(2):
        v0 = h * _VHALF
        vgs = pl.ds(h * (_VHALF // 8), _VHALF // 8)
        handles = [
            pltpu.async_copy(
                data_hbm.at[b, vgs, rt0, :, :],
                in_slab.at[:, :, pl.ds(16, 128)],
                sem_i,
            ),
            pltpu.async_copy(
                data_hbm.at[b, vgs, rt0 + 1, :, :],
                in_slab.at[:, :, pl.ds(144, 128)],
                sem_i,
            ),
            pltpu.async_copy(
                data_hbm.at[b, vgs, rtl, :, pl.ds(112, 16)],
                in_slab.at[:, :, pl.ds(0, 16)],
                sem_i,
            ),
            pltpu.async_copy(
                data_hbm.at[b, vgs, rtr, :, pl.ds(0, 16)],
                in_slab.at[:, :, pl.ds(272, 16)],
                sem_i,
            ),
        ]
        for hh in handles:
            hh.wait()

        def row_pair(vi, _):
            ga = vi // 4
            ra = 2 * (vi % 4)
            _m_pass2(in_slab, ga, ra, ra + 1, m_tile)
            _final_pass2(m_tile, os_tile, 10 + v0 + 2 * vi, 11 + v0 + 2 * vi)
            return 0

        lax.fori_loop(0, _VHALF // 2, row_pair, 0)

    for hr in range(10):
        for jb in range(_RCHUNK // 16):
            os_tile[hr, pl.ds(jb * 16, 16)] = os_tile[256 + hr, pl.ds(jb * 16, 16)]
    for hr in range(11):
        for jb in range(_RCHUNK // 16):
            os_tile[266 + hr, pl.ds(jb * 16, 16)] = os_tile[10 + hr, pl.ds(jb * 16, 16)]

    out_handles = []
    for g in range(4):
        vg = g * 64
        for hh in out_handles:
            hh.wait()
        out_handles = []

        def ca_col_block(jbq, _):
            cbs = [pl.ds((2 * jbq + jj) * 16, 16) for jj in range(2)]
            accs = []
            for cb in cbs:
                acc = None
                for off in _OFFS:
                    x = os_tile[vg + 10 + off, cb]
                    acc = x if acc is None else acc + x
                accs.append(acc)

            def ca_rows(vv, ss):
                v = vg + 4 * vv
                oq = vv // 2
                orr = 4 * (vv % 2)
                out = []
                for cb, s in zip(cbs, ss):
                    for u in range(4):
                        out_slab[oq, orr + u, cb] = s * _SCALE
                        d = (os_tile[v + u + 21, cb] + os_tile[v + u + 8, cb]) - (
                            os_tile[v + u, cb] + os_tile[v + u + 13, cb]
                        )
                        s = s + d
                    out.append(s)
                return tuple(out)

            lax.fori_loop(0, 16, ca_rows, tuple(accs))
            return 0

        lax.fori_loop(0, _RCHUNK // 32, ca_col_block, 0)
        for j in range(2):
            out_handles.append(
                pltpu.async_copy(
                    out_slab.at[:, :, pl.ds(128 * j, 128)],
                    out_hbm.at[b, pl.ds(g * 8, 8), rt0 + j, :, :],
                    sem_o,
                )
            )
    for hh in out_handles:
        hh.wait()


@jax.jit
def kernel(data):
    mesh = plsc.VectorSubcoreMesh(core_axis_name="c", subcore_axis_name="s")
    run = functools.partial(
        pl.kernel,
        mesh=mesh,
        out_type=jax.ShapeDtypeStruct((_B, _V // 8, _R // 128, 8, 128), jnp.float32),
        scratch_types=[
            pltpu.VMEM((_VHALF // 8, 8, _COLS), jnp.float32),
            pltpu.VMEM((_V + 21, _RCHUNK), jnp.float32),
            pltpu.VMEM((8, 8, _RCHUNK), jnp.float32),
            pltpu.VMEM((8, _COLS), jnp.float32),
            pltpu.SemaphoreType.DMA,
            pltpu.SemaphoreType.DMA,
        ],
        compiler_params=pltpu.CompilerParams(use_tc_tiling_on_sc=False),
    )(_cfar_body)
    data5 = data.reshape(_B, _V // 8, 8, _R // 128, 128).transpose(0, 1, 3, 2, 4)
    out5 = run(data5)
    return out5.transpose(0, 1, 3, 2, 4).reshape(_B, _V, _R)
